# baseline (speedup 1.0000x reference)
import functools

import jax
import jax.numpy as jnp
from jax.experimental import pallas as pl
from jax.experimental.pallas import tpu as pltpu

DEADE = -(1 << 28)
MANT_MASK = 0x007FFFFF
ONE_BITS = 0x3F800000
LN2HI = 0.69314575195
LN2LO = 1.42860677e-06


def _gather_kernel(lp_ref, cls_ref, w_ref):
    x = lp_ref[0]
    m = jnp.max(x, axis=1, keepdims=True)
    e = jnp.exp(x - m)
    z = jnp.sum(e, axis=1, keepdims=True)
    C = x.shape[1]
    cls = cls_ref[0]
    cidx = jax.lax.broadcasted_iota(jnp.int32, (C, 128), 0)
    oh = (cidx == cls).astype(jnp.bfloat16)
    g = jnp.dot(e.astype(jnp.bfloat16), oh,
                preferred_element_type=jnp.float32)
    w_ref[0] = g * (1.0 / z)


def _shift1(x, fill):
    b = x.shape[0]
    return jnp.concatenate([jnp.full((b, 1), fill, x.dtype), x[:, :-1]], axis=1)


def _shift2(x, fill):
    b = x.shape[0]
    return jnp.concatenate([jnp.full((b, 2), fill, x.dtype), x[:, :-2]], axis=1)


def _decomp(p):
    bits = jax.lax.bitcast_convert_type(p, jnp.int32)
    e = jax.lax.shift_right_logical(bits, 23) - 127
    m = jax.lax.bitcast_convert_type(
        jax.lax.bitwise_or(jax.lax.bitwise_and(bits, MANT_MASK), ONE_BITS),
        jnp.float32)
    return m, e


def _scale(d):
    return jax.lax.bitcast_convert_type(
        jax.lax.shift_left(jnp.maximum(d + 127, 0), 23), jnp.float32)


def _dp_kernel(w_ref, skip_ref, len_ref, selb_ref, sela_ref, out_ref,
               mA_r, eA_r, m0_r, e0_r, *, tb):
    i = pl.program_id(0)
    nt = pl.num_programs(0)
    b = skip_ref.shape[0]
    lane = jax.lax.broadcasted_iota(jnp.int32, (b, 128), 1)

    @pl.when(i == 0)
    def _init():
        p0 = w_ref[0]
        mi, ei = _decomp(p0)
        mA_r[...] = jnp.where(lane == 0, mi, 1.0)
        eA_r[...] = jnp.where(lane == 0, ei, DEADE)
        pb0 = jnp.broadcast_to(p0[:, 1:2], (b, 128))
        m0i, e0i = _decomp(pb0)
        m0_r[...] = m0i
        e0_r[...] = e0i

    skipm = skip_ref[...] > 0
    leni = len_ref[...]
    mA = mA_r[...]
    eA = eA_r[...]
    m0 = m0_r[...]
    e0 = e0_r[...]
    for tt in range(tb):
        t = i * tb + tt
        p = w_ref[tt]
        m1 = jnp.where(lane == 0, m0, _shift1(mA, 1.0))
        e1 = jnp.where(lane == 0, e0, _shift1(eA, DEADE))
        m2 = _shift2(mA, 1.0)
        e2 = jnp.where(skipm, _shift2(eA, DEADE), DEADE)
        E = jnp.maximum(jnp.maximum(eA, e1), e2)
        msum = (mA * _scale(eA - E) + m1 * _scale(e1 - E)
                + m2 * _scale(e2 - E)) * p
        bits = jax.lax.bitcast_convert_type(msum, jnp.int32)
        eb = jax.lax.shift_right_logical(bits, 23)
        mN = jax.lax.bitcast_convert_type(
            jax.lax.bitwise_or(jax.lax.bitwise_and(bits, MANT_MASK), ONE_BITS),
            jnp.float32)
        eN = E + (eb - 127)
        pb = jnp.broadcast_to(p[:, 1:2], (b, 128))
        m0n = m0 * pb
        bits0 = jax.lax.bitcast_convert_type(m0n, jnp.int32)
        m0n = jax.lax.bitcast_convert_type(
            jax.lax.bitwise_or(jax.lax.bitwise_and(bits0, MANT_MASK),
                               ONE_BITS), jnp.float32)
        e0n = e0 + (jax.lax.shift_right_logical(bits0, 23) - 127)
        act = (t < leni) & (t > 0)
        mA = jnp.where(act, mN, mA)
        eA = jnp.where(act, eN, eA)
        m0 = jnp.where(act, m0n, m0)
        e0 = jnp.where(act, e0n, e0)
    mA_r[...] = mA
    eA_r[...] = eA
    m0_r[...] = m0
    e0_r[...] = e0

    @pl.when(i == nt - 1)
    def _fin():
        selb = selb_ref[...] > 0
        sela = sela_ref[...] > 0
        mb = jnp.max(jnp.where(selb, mA, 0.0), axis=1, keepdims=True)
        ebx = jnp.max(jnp.where(selb, eA, DEADE), axis=1, keepdims=True)
        ma = jnp.max(jnp.where(sela, mA, 0.0), axis=1, keepdims=True)
        eax = jnp.max(jnp.where(sela, eA, DEADE), axis=1, keepdims=True)
        E2 = jnp.maximum(ebx, eax)
        v = mb * _scale(ebx - E2) + ma * _scale(eax - E2)
        e2f = E2.astype(jnp.float32)
        loss = -(jnp.log(v) + e2f * LN2HI + e2f * LN2LO)
        out_ref[...] = jnp.broadcast_to(loss, (b, 128))


@jax.jit
def kernel(log_probs, targets, input_lengths, target_lengths):
    B, T, C = log_probs.shape
    L = targets.shape[1]
    targets = targets.astype(jnp.int32)
    input_lengths = input_lengths.astype(jnp.int32)
    target_lengths = target_lengths.astype(jnp.int32)

    lane = jnp.arange(128, dtype=jnp.int32)[None, :]
    lab_of_lane = jnp.clip(lane // 2, 0, L - 1)
    tgt_at_lane = jnp.take_along_axis(
        targets, jnp.broadcast_to(lab_of_lane, (B, 128)), axis=1)
    cls = jnp.where(lane % 2 == 0, tgt_at_lane, 0)
    cls = cls[:, None, :]
    prev_at_lane = jnp.take_along_axis(
        targets, jnp.broadcast_to(jnp.clip(lane // 2 - 1, 0, L - 1), (B, 128)),
        axis=1)
    skip = (lane % 2 == 0) & (lane >= 2) & (tgt_at_lane != prev_at_lane)
    skipf = skip.astype(jnp.float32)
    lenb = jnp.broadcast_to(input_lengths[:, None], (B, 128))
    selb = (lane == 2 * target_lengths[:, None] - 1).astype(jnp.float32)
    sela = (lane == 2 * target_lengths[:, None] - 2).astype(jnp.float32)

    w = pl.pallas_call(
        _gather_kernel,
        grid=(B,),
        in_specs=[
            pl.BlockSpec((1, T, C), lambda i: (i, 0, 0)),
            pl.BlockSpec((1, 1, 128), lambda i: (i, 0, 0)),
        ],
        out_specs=pl.BlockSpec((1, T, 128), lambda i: (i, 0, 0)),
        out_shape=jax.ShapeDtypeStruct((B, T, 128), jnp.float32),
        compiler_params=pltpu.CompilerParams(
            dimension_semantics=("arbitrary",)),
    )(log_probs, cls)

    wt = jnp.transpose(w, (1, 0, 2))

    TB = 64
    NT = T // TB
    out = pl.pallas_call(
        functools.partial(_dp_kernel, tb=TB),
        grid=(NT,),
        in_specs=[
            pl.BlockSpec((TB, B, 128), lambda i: (i, 0, 0)),
            pl.BlockSpec((B, 128), lambda i: (0, 0)),
            pl.BlockSpec((B, 128), lambda i: (0, 0)),
            pl.BlockSpec((B, 128), lambda i: (0, 0)),
            pl.BlockSpec((B, 128), lambda i: (0, 0)),
        ],
        out_specs=pl.BlockSpec((B, 128), lambda i: (0, 0)),
        out_shape=jax.ShapeDtypeStruct((B, 128), jnp.float32),
        scratch_shapes=[
            pltpu.VMEM((B, 128), jnp.float32),
            pltpu.VMEM((B, 128), jnp.int32),
            pltpu.VMEM((B, 128), jnp.float32),
            pltpu.VMEM((B, 128), jnp.int32),
        ],
        compiler_params=pltpu.CompilerParams(
            dimension_semantics=("arbitrary",)),
    )(wt, skipf, lenb, selb, sela)

    return out[:, 0]
